# baseline (speedup 1.0000x reference)
import functools

import jax
import jax.numpy as jnp
from jax import lax
from jax.experimental import pallas as pl
from jax.experimental.pallas import tpu as pltpu
from jax.experimental.pallas import tpu_sc as plsc

D = 64
B = 16384
H = 50
NC, NS = 2, 16
NW = NC * NS
BW = B // NW
Q = 128
NQ = BW // Q

_mesh = plsc.VectorSubcoreMesh(core_axis_name="c", subcore_axis_name="s")


@functools.partial(
    pl.kernel,
    mesh=_mesh,
    out_type=jax.ShapeDtypeStruct((H, D, B), jnp.float32),
    compiler_params=pltpu.CompilerParams(use_tc_tiling_on_sc=True, needs_layout_passes=False),
    scratch_types=[
        pltpu.VMEM((BW,), jnp.int32),
        pltpu.VMEM((BW,), jnp.int32),
        pltpu.VMEM((Q, 128), jnp.float32),
        pltpu.VMEM((D, Q), jnp.float32),
        pltpu.SemaphoreType.DMA,
    ],
)
def _gather_kernel(idx2_hbm, col_hbm, tab_hbm, out_hbm, idx2_v, col_v, gbuf, obuf, gsem):
    wid = lax.axis_index("s") * NC + lax.axis_index("c")
    b0 = wid * BW

    def per_h(h, carry):
        pltpu.sync_copy(idx2_hbm.at[h, pl.ds(b0, BW)], idx2_v)
        pltpu.sync_copy(col_hbm.at[h, pl.ds(b0, BW)], col_v)
        for q in range(NQ):
            pltpu.async_copy(
                tab_hbm.at[idx2_v.at[pl.ds(q * Q, Q)]], gbuf, gsem
            ).wait()

            def per_k(k, c2):
                rows = lax.iota(jnp.int32, 16) + k * 16
                cbase = col_v[pl.ds(q * Q + k * 16, 16)]
                for f in range(D):
                    vals = plsc.load_gather(gbuf, [rows, cbase + f])
                    obuf[f, pl.ds(k * 16, 16)] = vals
                return c2

            lax.fori_loop(0, Q // 16, per_k, 0)
            pltpu.sync_copy(obuf, out_hbm.at[h, :, pl.ds(b0 + q * Q, Q)])
        return carry

    lax.fori_loop(0, H, per_h, 0)


def kernel(indices, embeddings):
    idx_t = indices.T.astype(jnp.int32)
    idx2 = idx_t >> 1
    col = (idx_t & 1) << 6
    tab2 = embeddings.reshape(500000, 128)
    out3 = _gather_kernel(idx2, col, tab2)
    return jnp.transpose(out3, (2, 0, 1))


# speedup vs baseline: 1.1669x; 1.1669x over previous
import functools

import jax
import jax.numpy as jnp
from jax import lax
from jax.experimental import pallas as pl
from jax.experimental.pallas import tpu as pltpu
from jax.experimental.pallas import tpu_sc as plsc

D = 64
B = 16384
H = 50
NC, NS = 2, 16
NW = NC * NS
BW = B // NW
Q = 128
NQ = BW // Q

_mesh = plsc.VectorSubcoreMesh(core_axis_name="c", subcore_axis_name="s")


@functools.partial(
    pl.kernel,
    mesh=_mesh,
    out_type=jax.ShapeDtypeStruct((H, D, B), jnp.float32),
    compiler_params=pltpu.CompilerParams(
        use_tc_tiling_on_sc=True,
        needs_layout_passes=False,
        disable_bounds_checks=True,
    ),
    scratch_types=[
        [pltpu.VMEM((BW,), jnp.int32) for _ in range(2)],
        [pltpu.VMEM((BW,), jnp.int32) for _ in range(2)],
        [pltpu.VMEM((Q, 128), jnp.float32) for _ in range(2)],
        [pltpu.VMEM((D, Q), jnp.float32) for _ in range(2)],
        [pltpu.SemaphoreType.DMA for _ in range(2)],
        [pltpu.SemaphoreType.DMA for _ in range(2)],
        [pltpu.SemaphoreType.DMA for _ in range(2)],
    ],
)
def _gather_kernel(idx2_hbm, col_hbm, tab_hbm, out_hbm,
                   i2, cl, gbuf, obuf, isem, gsem, ssem):
    wid = lax.axis_index("s") * NC + lax.axis_index("c")
    b0 = wid * BW

    def fire_idx(ib, h):
        pltpu.async_copy(idx2_hbm.at[h, pl.ds(b0, BW)], i2[ib], isem[ib])
        pltpu.async_copy(col_hbm.at[h, pl.ds(b0, BW)], cl[ib], isem[ib])

    def wait_idx(ib):
        pltpu.make_async_copy(idx2_hbm.at[0, pl.ds(b0, BW)], i2[ib], isem[ib]).wait()
        pltpu.make_async_copy(col_hbm.at[0, pl.ds(b0, BW)], cl[ib], isem[ib]).wait()

    def fire_g(qb, ib, q):
        pltpu.async_copy(tab_hbm.at[i2[ib].at[pl.ds(q * Q, Q)]], gbuf[qb], gsem[qb])

    def wait_g(qb):
        pltpu.make_async_copy(
            tab_hbm.at[i2[0].at[pl.ds(0, Q)]], gbuf[qb], gsem[qb]
        ).wait()

    def fire_s(qb, h, q):
        pltpu.async_copy(obuf[qb], out_hbm.at[h, :, pl.ds(b0 + q * Q, Q)], ssem[qb])

    def wait_s(qb):
        pltpu.make_async_copy(
            obuf[qb], out_hbm.at[0, :, pl.ds(b0, Q)], ssem[qb]
        ).wait()

    def transpose(ib, qb, q):
        def per_k(k, c2):
            rows = lax.iota(jnp.int32, 16) + k * 16
            cbase = cl[ib][pl.ds(q * Q + k * 16, 16)]

            def per_fc(fc, c3):
                for df in range(8):
                    f = fc * 8 + df
                    vals = plsc.load_gather(gbuf[qb], [rows, cbase + f])
                    obuf[qb][f, pl.ds(k * 16, 16)] = vals
                return c3

            lax.fori_loop(0, D // 8, per_fc, 0)
            return c2

        lax.fori_loop(0, Q // 16, per_k, 0)

    def do_h(h, ib, first_h, last_h):
        wait_idx(ib)
        if not last_h:
            fire_idx(1 - ib, h + 1)
        fire_g(0, ib, 0)
        for q in range(NQ):
            qb = q % 2
            wait_g(qb)
            if q < NQ - 1:
                fire_g(1 - qb, ib, q + 1)
            if not (first_h and q < 2):
                wait_s(qb)
            transpose(ib, qb, q)
            fire_s(qb, h, q)

    fire_idx(0, 0)
    do_h(0, 0, True, False)
    do_h(1, 1, False, False)

    def h_group(g, carry):
        do_h(2 * g, 0, False, False)
        do_h(2 * g + 1, 1, False, False)
        return carry

    lax.fori_loop(1, H // 2 - 1, h_group, 0)

    do_h(H - 2, 0, False, False)
    do_h(H - 1, 1, False, True)
    wait_s(0)
    wait_s(1)


def kernel(indices, embeddings):
    idx_t = indices.T.astype(jnp.int32)
    idx2 = idx_t >> 1
    col = (idx_t & 1) << 6
    tab2 = embeddings.reshape(500000, 128)
    out3 = _gather_kernel(idx2, col, tab2)
    return jnp.transpose(out3, (2, 0, 1))


# speedup vs baseline: 2.0115x; 1.7238x over previous
import functools

import jax
import jax.numpy as jnp
from jax import lax
from jax.experimental import pallas as pl
from jax.experimental.pallas import tpu as pltpu
from jax.experimental.pallas import tpu_sc as plsc

D = 64
B = 16384
H = 50
NC, NS = 2, 16
NW = NC * NS
BW = B // NW
Q = 128
NQ = BW // Q

_mesh = plsc.VectorSubcoreMesh(core_axis_name="c", subcore_axis_name="s")


@functools.partial(
    pl.kernel,
    mesh=_mesh,
    out_type=jax.ShapeDtypeStruct((H, D, B), jnp.float32),
    compiler_params=pltpu.CompilerParams(
        use_tc_tiling_on_sc=True,
        needs_layout_passes=False,
        disable_bounds_checks=True,
    ),
    scratch_types=[
        [pltpu.VMEM((BW,), jnp.int32) for _ in range(2)],
        [pltpu.VMEM((Q, 128), jnp.float32) for _ in range(2)],
        [pltpu.VMEM((D, Q), jnp.float32) for _ in range(2)],
        [pltpu.SemaphoreType.DMA for _ in range(2)],
        [pltpu.SemaphoreType.DMA for _ in range(2)],
        [pltpu.SemaphoreType.DMA for _ in range(2)],
    ],
)
def _gather_kernel(idx2_hbm, tab_hbm, out_hbm,
                   i2, gbuf, obuf, isem, gsem, ssem):
    wid = lax.axis_index("s") * NC + lax.axis_index("c")
    b0 = wid * BW

    def fire_idx(ib, h):
        pltpu.async_copy(idx2_hbm.at[h, pl.ds(b0, BW)], i2[ib], isem[ib])

    def wait_idx(ib):
        pltpu.make_async_copy(idx2_hbm.at[0, pl.ds(b0, BW)], i2[ib], isem[ib]).wait()

    def fire_g(qb, ib, q):
        pltpu.async_copy(tab_hbm.at[i2[ib].at[pl.ds(q * Q, Q)]], gbuf[qb], gsem[qb])

    def wait_g(qb):
        pltpu.make_async_copy(
            tab_hbm.at[i2[0].at[pl.ds(0, Q)]], gbuf[qb], gsem[qb]
        ).wait()

    def fire_s(qb, h, q):
        pltpu.async_copy(obuf[qb], out_hbm.at[h, :, pl.ds(b0 + q * Q, Q)], ssem[qb])

    def wait_s(qb):
        pltpu.make_async_copy(
            obuf[qb], out_hbm.at[0, :, pl.ds(b0, Q)], ssem[qb]
        ).wait()

    def transpose(ib, qb, q):
        def per_k(k, c2):
            rows = lax.iota(jnp.int32, 16) + k * 16

            @plsc.parallel_loop(0, D, unroll=8)
            def per_f(f):
                vals = plsc.load_gather(gbuf[qb], [rows, jnp.full((16,), f, jnp.int32)])
                obuf[qb][f, pl.ds(k * 16, 16)] = vals

            return c2

        lax.fori_loop(0, Q // 16, per_k, 0)

    def do_h(h, ib, first_h, last_h):
        wait_idx(ib)
        if not last_h:
            fire_idx(1 - ib, h + 1)
        fire_g(0, ib, 0)
        for q in range(NQ):
            qb = q % 2
            wait_g(qb)
            if q < NQ - 1:
                fire_g(1 - qb, ib, q + 1)
            if not (first_h and q < 2):
                wait_s(qb)
            transpose(ib, qb, q)
            fire_s(qb, h, q)

    fire_idx(0, 0)
    do_h(0, 0, True, False)
    do_h(1, 1, False, False)

    def h_group(g, carry):
        do_h(2 * g, 0, False, False)
        do_h(2 * g + 1, 1, False, False)
        return carry

    lax.fori_loop(1, H // 2 - 1, h_group, 0)

    do_h(H - 2, 0, False, False)
    do_h(H - 1, 1, False, True)
    wait_s(0)
    wait_s(1)


def kernel(indices, embeddings):
    idx_t = indices.T.astype(jnp.int32)
    tab2 = jnp.pad(embeddings, ((0, 0), (0, 64)))
    out3 = _gather_kernel(idx_t, tab2)
    return jnp.transpose(out3, (2, 0, 1))
